# baseline (speedup 1.0000x reference)
import functools

import jax
import jax.numpy as jnp
from jax import lax
from jax.experimental import pallas as pl
from jax.experimental.pallas import tpu as pltpu
from jax.experimental.pallas import tpu_sc as plsc

N_CTX = 8192
N_STATE = 1024


_ROWS_PER_BLOCK = 256


def _normalize_body(emb_ref, out_ref):
    x = emb_ref[...]
    norm = jnp.sqrt(jnp.sum(x * x, axis=-1, keepdims=True))
    out_ref[...] = x * (1.0 / jnp.maximum(norm, 1e-12))


def _normalize_table(table):
    n_rows, d = table.shape
    grid = (n_rows // _ROWS_PER_BLOCK,)
    return pl.pallas_call(
        _normalize_body,
        grid=grid,
        in_specs=[pl.BlockSpec((_ROWS_PER_BLOCK, d), lambda i: (i, 0))],
        out_specs=pl.BlockSpec((_ROWS_PER_BLOCK, d), lambda i: (i, 0)),
        out_shape=jax.ShapeDtypeStruct((n_rows, d), table.dtype),
    )(table)


_NC = 2
_NS = 16
_NW = _NC * _NS

_CA = 64
_CB = 56
_PAIR = _CA + _CB


def _make_gather(B, D):
    b_per_w = B // _NW
    n_pairs = (b_per_w - _CA) // _PAIR
    assert b_per_w == n_pairs * _PAIR + _CA
    mesh = plsc.VectorSubcoreMesh(core_axis_name="c", subcore_axis_name="s")

    @functools.partial(
        pl.kernel,
        mesh=mesh,
        out_type=jax.ShapeDtypeStruct((B, D), jnp.float32),
        scratch_types=[
            pltpu.VMEM((b_per_w,), jnp.int32),
            pltpu.VMEM((_CA, D), jnp.float32),
            pltpu.VMEM((_CB, D), jnp.float32),
            pltpu.SemaphoreType.DMA,
            pltpu.SemaphoreType.DMA,
        ],
    )
    def gather(table_hbm, idx_hbm, out_hbm, idx_v, buf_a, buf_b, sem_a, sem_b):
        wid = lax.axis_index("s") * _NC + lax.axis_index("c")
        base = wid * b_per_w
        pltpu.sync_copy(idx_hbm.at[pl.ds(base, b_per_w)], idx_v)

        def fire_a(start):
            pltpu.async_copy(
                table_hbm.at[idx_v.at[pl.ds(start, _CA)]], buf_a, sem_a)

        def fire_b(start):
            pltpu.async_copy(
                table_hbm.at[idx_v.at[pl.ds(start, _CB)]], buf_b, sem_b)

        def wait_a():
            pltpu.make_async_copy(
                table_hbm.at[pl.ds(0, _CA)], buf_a, sem_a).wait()

        def wait_b():
            pltpu.make_async_copy(
                table_hbm.at[pl.ds(0, _CB)], buf_b, sem_b).wait()

        fire_a(0)

        def body(m, _):
            st = pl.multiple_of(m * _PAIR, 8)
            fire_b(st + _CA)
            wait_a()
            pltpu.sync_copy(
                buf_a, out_hbm.at[pl.ds(base + st, _CA)])
            fire_a(st + _PAIR)
            wait_b()
            pltpu.sync_copy(
                buf_b, out_hbm.at[pl.ds(base + st + _CA, _CB)])
            return 0

        lax.fori_loop(0, n_pairs, body, 0)

        wait_a()
        pltpu.sync_copy(
            buf_a, out_hbm.at[pl.ds(base + n_pairs * _PAIR, _CA)])

    return gather


def kernel(positions, positional_embeddings):
    bsz, seq = positions.shape
    n_rows, d = positional_embeddings.shape
    normed = _normalize_table(positional_embeddings)
    idx = positions.reshape(-1).astype(jnp.int32)
    out = _make_gather(bsz * seq, d)(normed, idx)
    return out.reshape(bsz, seq, d)


# speedup vs baseline: 1.0924x; 1.0924x over previous
import functools

import jax
import jax.numpy as jnp
from jax import lax
from jax.experimental import pallas as pl
from jax.experimental.pallas import tpu as pltpu
from jax.experimental.pallas import tpu_sc as plsc

N_CTX = 8192
N_STATE = 1024


_ROWS_PER_BLOCK = 2048


def _normalize_body(emb_ref, out_ref):
    x = emb_ref[...]
    norm = jnp.sqrt(jnp.sum(x * x, axis=-1, keepdims=True))
    out_ref[...] = x * (1.0 / jnp.maximum(norm, 1e-12))


def _normalize_table(table):
    n_rows, d = table.shape
    grid = (n_rows // _ROWS_PER_BLOCK,)
    return pl.pallas_call(
        _normalize_body,
        grid=grid,
        in_specs=[pl.BlockSpec((_ROWS_PER_BLOCK, d), lambda i: (i, 0))],
        out_specs=pl.BlockSpec((_ROWS_PER_BLOCK, d), lambda i: (i, 0)),
        out_shape=jax.ShapeDtypeStruct((n_rows, d), table.dtype),
    )(table)


_NC = 2
_NS = 16
_NW = _NC * _NS

_CA = 64
_CB = 56
_PAIR = _CA + _CB


def _make_gather(B, D):
    b_per_w = B // _NW
    n_pairs = (b_per_w - _CA) // _PAIR
    assert b_per_w == n_pairs * _PAIR + _CA
    mesh = plsc.VectorSubcoreMesh(core_axis_name="c", subcore_axis_name="s")

    @functools.partial(
        pl.kernel,
        mesh=mesh,
        out_type=jax.ShapeDtypeStruct((B, D), jnp.float32),
        scratch_types=[
            pltpu.VMEM((b_per_w,), jnp.int32),
            pltpu.VMEM((_CA, D), jnp.float32),
            pltpu.VMEM((_CB, D), jnp.float32),
            pltpu.SemaphoreType.DMA,
            pltpu.SemaphoreType.DMA,
        ],
    )
    def gather(table_hbm, idx_hbm, out_hbm, idx_v, buf_a, buf_b, sem_a, sem_b):
        wid = lax.axis_index("s") * _NC + lax.axis_index("c")
        base = wid * b_per_w
        pltpu.sync_copy(idx_hbm.at[pl.ds(base, b_per_w)], idx_v)

        def fire_a(start):
            pltpu.async_copy(
                table_hbm.at[idx_v.at[pl.ds(start, _CA)]], buf_a, sem_a)

        def fire_b(start):
            pltpu.async_copy(
                table_hbm.at[idx_v.at[pl.ds(start, _CB)]], buf_b, sem_b)

        def wait_a():
            pltpu.make_async_copy(
                table_hbm.at[pl.ds(0, _CA)], buf_a, sem_a).wait()

        def wait_b():
            pltpu.make_async_copy(
                table_hbm.at[pl.ds(0, _CB)], buf_b, sem_b).wait()

        fire_a(0)

        def body(m, _):
            st = pl.multiple_of(m * _PAIR, 8)
            fire_b(st + _CA)
            wait_a()
            pltpu.sync_copy(
                buf_a, out_hbm.at[pl.ds(base + st, _CA)])
            fire_a(st + _PAIR)
            wait_b()
            pltpu.sync_copy(
                buf_b, out_hbm.at[pl.ds(base + st + _CA, _CB)])
            return 0

        lax.fori_loop(0, n_pairs, body, 0)

        wait_a()
        pltpu.sync_copy(
            buf_a, out_hbm.at[pl.ds(base + n_pairs * _PAIR, _CA)])

    return gather


def kernel(positions, positional_embeddings):
    bsz, seq = positions.shape
    n_rows, d = positional_embeddings.shape
    normed = _normalize_table(positional_embeddings)
    idx = positions.reshape(-1).astype(jnp.int32)
    out = _make_gather(bsz * seq, d)(normed, idx)
    return out.reshape(bsz, seq, d)
